# Initial kernel scaffold; baseline (speedup 1.0000x reference)
import jax
import jax.numpy as jnp
from jax.experimental import pallas as pl


def kernel(x, router_w, w1_all, b1_all, w2_all, b2_all):
    raise NotImplementedError("write your pallas kernel here")


# speedup vs baseline: 22.7212x; 22.7212x over previous
import jax
import jax.numpy as jnp
from jax.experimental import pallas as pl

B, N, C, FF, E, K = 16, 8, 256, 1024, 8, 2
L = B * N


def _moe_kernel(x_ref, rw_ref, w1_ref, b1_ref, w2_ref, b2_ref, out_ref):
    xf = x_ref[:]
    logits = jax.lax.dot_general(
        xf, rw_ref[:], dimension_numbers=(((1,), (1,)), ((), ())),
        preferred_element_type=jnp.float32)
    m = jnp.max(logits, axis=1, keepdims=True)
    ex = jnp.exp(logits - m)
    probs = ex / jnp.sum(ex, axis=1, keepdims=True)

    col = jax.lax.broadcasted_iota(jnp.int32, (L, E), 1)
    p1 = jnp.max(probs, axis=1, keepdims=True)
    i1 = jnp.min(jnp.where(probs == p1, col, E), axis=1, keepdims=True)
    mask1 = col == i1
    pm = jnp.where(mask1, -1.0, probs)
    p2 = jnp.max(pm, axis=1, keepdims=True)
    i2 = jnp.min(jnp.where(pm == p2, col, E), axis=1, keepdims=True)
    mask2 = col == i2
    denom = p1 + p2 + 1e-9
    gates = (jnp.where(mask1, probs, 0.0) + jnp.where(mask2, probs, 0.0)) / denom

    acc = jnp.zeros((L, C), dtype=jnp.float32)
    for e in range(E):
        h = jax.lax.dot_general(
            xf, w1_ref[e], dimension_numbers=(((1,), (1,)), ((), ())),
            preferred_element_type=jnp.float32) + b1_ref[e][None, :]
        h = jnp.maximum(h, 0.0)
        o = jax.lax.dot_general(
            h, w2_ref[e], dimension_numbers=(((1,), (1,)), ((), ())),
            preferred_element_type=jnp.float32) + b2_ref[e][None, :]
        acc = acc + gates[:, e:e + 1] * o
    out_ref[:] = acc


def kernel(x, router_w, w1_all, b1_all, w2_all, b2_all):
    xf = x.reshape(L, C)
    out = pl.pallas_call(
        _moe_kernel,
        out_shape=jax.ShapeDtypeStruct((L, C), jnp.float32),
    )(xf, router_w, w1_all, b1_all, w2_all, b2_all)
    return out.reshape(B, N, C)
